# Initial kernel scaffold; baseline (speedup 1.0000x reference)
import jax
import jax.numpy as jnp
from jax.experimental import pallas as pl


def kernel(x, edge_index, edge_weight, W1, b1, g1, be1, W2, b2, g2, be2, W3, b3, g3, be3, Wh1, bh1, Wh2, bh2):
    raise NotImplementedError("write your pallas kernel here")


# speedup vs baseline: 4.8303x; 4.8303x over previous
import functools

import jax
import jax.numpy as jnp
from jax import lax
from jax.experimental import pallas as pl
from jax.experimental.pallas import tpu as pltpu
from jax.experimental.pallas import tpu_sc as plsc

N = 10000
E = 160000
EP = 163840
NC = 2
NS = 16
CH = 128

@functools.lru_cache(maxsize=None)
def _mesh():
    return plsc.VectorSubcoreMesh(
        core_axis_name="c", subcore_axis_name="s",
        num_cores=NC, num_subcores=NS)


_Z16F = functools.partial(jnp.zeros, (16,), jnp.float32)


@functools.lru_cache(maxsize=None)
def _make_deg():
    return functools.partial(
        pl.kernel,
        out_type=jax.ShapeDtypeStruct((NC * NS, N), jnp.float32),
        mesh=_mesh(),
        compiler_params=pltpu.CompilerParams(needs_layout_passes=False),
        scratch_types=[
            pltpu.VMEM((N,), jnp.float32),
            pltpu.VMEM((CH,), jnp.int32),
            pltpu.VMEM((CH,), jnp.float32),
        ],
    )(_deg_body)


def _deg_body(col_hbm, ew_hbm, out_hbm, degl, colb, ewb):
    c = lax.axis_index("c")
    s = lax.axis_index("s")
    wid = s * NC + c
    per_tile = EP // (NC * NS)

    def zero(i, carry):
        degl[pl.ds(i * 16, 16)] = _Z16F()
        return carry

    lax.fori_loop(0, N // 16, zero, 0)

    def chunk(k, carry):
        off = wid * per_tile + k * CH
        pltpu.sync_copy(col_hbm.at[pl.ds(off, CH)], colb)
        pltpu.sync_copy(ew_hbm.at[pl.ds(off, CH)], ewb)
        for j in range(CH // 16):
            idx = colb[pl.ds(j * 16, 16)]
            w = ewb[pl.ds(j * 16, 16)]
            plsc.addupdate_scatter(degl, [idx], w)
        return carry

    lax.fori_loop(0, per_tile // CH, chunk, 0)
    pltpu.sync_copy(degl, out_hbm.at[wid])


@functools.lru_cache(maxsize=None)
def _make_msg(H, feat_split):
    rows_per_tile = N // NS
    zrows = 125
    per_tile = EP // NS if feat_split else EP // (NC * NS)

    @functools.partial(
        pl.kernel,
        out_type=jax.ShapeDtypeStruct((NC, N, H), jnp.float32),
        mesh=_mesh(),
        compiler_params=pltpu.CompilerParams(needs_layout_passes=False),
        scratch_types=[
            pltpu.VMEM_SHARED((N, H), jnp.float32),
            pltpu.VMEM((zrows, H), jnp.float32),
            pltpu.VMEM((CH,), jnp.int32),
            pltpu.VMEM((1, CH), jnp.int32),
            pltpu.VMEM((CH,), jnp.float32),
            pltpu.VMEM((CH,), jnp.int32),
            pltpu.VMEM((CH, H), jnp.float32),
            pltpu.SemaphoreType.DMA,
        ],
    )
    def msg(y_hbm, row_hbm, col_hbm, ew_hbm, out_hbm,
            acc, zbuf, rowb, colb, ewb, gidx, rows, sem):
        c = lax.axis_index("c")
        s = lax.axis_index("s")

        def zero(i, carry):
            for j in range(H // 16):
                zbuf[i, pl.ds(j * 16, 16)] = _Z16F()
            return carry

        lax.fori_loop(0, zrows, zero, 0)
        for j in range(rows_per_tile // zrows):
            pltpu.sync_copy(
                zbuf, acc.at[pl.ds(s * rows_per_tile + j * zrows, zrows)])
        plsc.subcore_barrier()

        tile0 = s * per_tile if feat_split else (c * NS + s) * per_tile

        def chunk(k, carry):
            off = tile0 + k * CH
            pltpu.sync_copy(row_hbm.at[pl.ds(off, CH)], rowb)
            pltpu.sync_copy(col_hbm.at[pl.ds(off, CH)], colb.at[0])
            pltpu.sync_copy(ew_hbm.at[pl.ds(off, CH)], ewb)
            for j in range(CH // 16):
                r = rowb[pl.ds(j * 16, 16)]
                gidx[pl.ds(j * 16, 16)] = r * 2 + c if feat_split else r
            pltpu.async_copy(y_hbm.at[gidx], rows, sem).wait()

            def grp(g, gcarry):
                w = ewb[pl.ds(g * 16, 16)]
                for l in range(16):
                    f = w[l]
                    e = g * 16 + l
                    for j in range(H // 16):
                        rows[e, pl.ds(j * 16, 16)] = (
                            rows[e, pl.ds(j * 16, 16)] * f)
                return gcarry

            lax.fori_loop(0, CH // 16, grp, 0)
            pltpu.sync_copy(rows, acc.at[colb.at[0]], add=True)
            return carry

        lax.fori_loop(0, per_tile // CH, chunk, 0)
        plsc.subcore_barrier()

        @pl.when(s == 0)
        def _copy_out():
            pltpu.sync_copy(acc, out_hbm.at[c])

    return msg


def _tc_first_body(degp_ref, x_ref, w_ref, dis_ref, y_ref):
    deg = jnp.sum(degp_ref[...], axis=1, keepdims=True) + 1.0
    dis = lax.rsqrt(deg)
    dis_ref[...] = dis
    y_ref[...] = jnp.dot(x_ref[...], w_ref[...],
                         preferred_element_type=jnp.float32) * dis


def _combine_body(acc_ref, y_ref, dis_ref, b_ref, g_ref, be_ref, w_ref,
                  out_ref):
    sm = jnp.concatenate([acc_ref[0], acc_ref[1]], axis=1) + y_ref[...]
    dis = dis_ref[...]
    o = sm * dis + b_ref[...]
    mu = jnp.mean(o, axis=0, keepdims=True)
    var = jnp.mean(o * o, axis=0, keepdims=True) - mu * mu
    h = jnp.maximum(
        (o - mu) * lax.rsqrt(var + 1e-5) * g_ref[...] + be_ref[...], 0.0)
    out_ref[...] = jnp.dot(h, w_ref[...],
                           preferred_element_type=jnp.float32) * dis


def _final_body(acc_ref, y_ref, dis_ref, b_ref, g_ref, be_ref,
                wh1_ref, bh1_ref, wh2_ref, bh2_ref, logit_ref, z_ref):
    sm = acc_ref[0] + acc_ref[1] + y_ref[...]
    o = sm * dis_ref[...] + b_ref[...]
    mu = jnp.mean(o, axis=0, keepdims=True)
    var = jnp.mean(o * o, axis=0, keepdims=True) - mu * mu
    z = jnp.maximum(
        (o - mu) * lax.rsqrt(var + 1e-5) * g_ref[...] + be_ref[...], 0.0)
    z_ref[...] = z
    t = jnp.maximum(
        jnp.dot(z, wh1_ref[...], preferred_element_type=jnp.float32)
        + bh1_ref[...], 0.0)
    logit_ref[...] = (
        jnp.dot(t, wh2_ref[...], preferred_element_type=jnp.float32)
        + bh2_ref[...])


def _vmem_call(body, n_in, out_shapes):
    return pl.pallas_call(
        body,
        in_specs=[pl.BlockSpec(memory_space=pltpu.VMEM)] * n_in,
        out_specs=jax.tree.map(
            lambda _: pl.BlockSpec(memory_space=pltpu.VMEM), out_shapes),
        out_shape=out_shapes,
    )


_SDS = jax.ShapeDtypeStruct
_tc_first = _vmem_call(
    _tc_first_body, 3,
    (_SDS((N, 1), jnp.float32), _SDS((N, 256), jnp.float32)))
_tc_comb1 = _vmem_call(_combine_body, 7, _SDS((N, 256), jnp.float32))
_tc_comb2 = _vmem_call(_combine_body, 7, _SDS((N, 128), jnp.float32))
_tc_final = _vmem_call(
    _final_body, 10,
    (_SDS((N, 1), jnp.float32), _SDS((N, 128), jnp.float32)))


def kernel(x, edge_index, edge_weight, W1, b1, g1, be1, W2, b2, g2, be2,
           W3, b3, g3, be3, Wh1, bh1, Wh2, bh2):
    row = edge_index[0]
    col = edge_index[1]
    padi = jnp.zeros((EP - E,), edge_index.dtype)
    row_p = jnp.concatenate([row, padi])
    col_p = jnp.concatenate([col, padi])
    ew_p = jnp.concatenate([edge_weight, jnp.zeros((EP - E,), jnp.float32)])

    degp = _make_deg()(col_p, ew_p)
    dis, y1 = _tc_first(degp.T, x, W1)

    r2 = lambda a: a.reshape(1, -1)
    _msg_f = _make_msg(128, True)
    acc1 = _msg_f(y1.reshape(2 * N, 128), row_p, col_p, ew_p)
    y2 = _tc_comb1(acc1, y1, dis, r2(b1), r2(g1), r2(be1), W2)
    acc2 = _msg_f(y2.reshape(2 * N, 128), row_p, col_p, ew_p)
    y3 = _tc_comb2(acc2, y2, dis, r2(b2), r2(g2), r2(be2), W3)
    acc3 = _make_msg(128, False)(y3, row_p, col_p, ew_p)
    logits, z = _tc_final(acc3, y3, dis, r2(b3), r2(g3), r2(be3),
                          Wh1, r2(bh1), Wh2, r2(bh2))
    return logits.reshape(-1), z


# speedup vs baseline: 6.9850x; 1.4461x over previous
import functools

import jax
import jax.numpy as jnp
from jax import lax
from jax.experimental import pallas as pl
from jax.experimental.pallas import tpu as pltpu
from jax.experimental.pallas import tpu_sc as plsc

N = 10000
E = 160000
EP = 163840
NC = 2
NS = 16
CH = 128

@functools.lru_cache(maxsize=None)
def _mesh():
    return plsc.VectorSubcoreMesh(
        core_axis_name="c", subcore_axis_name="s",
        num_cores=NC, num_subcores=NS)


_Z16F = functools.partial(jnp.zeros, (16,), jnp.float32)


@functools.lru_cache(maxsize=None)
def _make_deg():
    return functools.partial(
        pl.kernel,
        out_type=jax.ShapeDtypeStruct((NC * NS, N), jnp.float32),
        mesh=_mesh(),
        compiler_params=pltpu.CompilerParams(needs_layout_passes=False),
        scratch_types=[
            pltpu.VMEM((N,), jnp.float32),
            pltpu.VMEM((CH,), jnp.int32),
            pltpu.VMEM((CH,), jnp.float32),
        ],
    )(_deg_body)


def _deg_body(col_hbm, ew_hbm, out_hbm, degl, colb, ewb):
    c = lax.axis_index("c")
    s = lax.axis_index("s")
    wid = s * NC + c
    per_tile = EP // (NC * NS)

    def zero(i, carry):
        degl[pl.ds(i * 16, 16)] = _Z16F()
        return carry

    lax.fori_loop(0, N // 16, zero, 0)

    def chunk(k, carry):
        off = wid * per_tile + k * CH
        pltpu.sync_copy(col_hbm.at[pl.ds(off, CH)], colb)
        pltpu.sync_copy(ew_hbm.at[pl.ds(off, CH)], ewb)
        for j in range(CH // 16):
            idx = colb[pl.ds(j * 16, 16)]
            w = ewb[pl.ds(j * 16, 16)]
            plsc.addupdate_scatter(degl, [idx], w)
        return carry

    lax.fori_loop(0, per_tile // CH, chunk, 0)
    pltpu.sync_copy(degl, out_hbm.at[wid])


@functools.lru_cache(maxsize=None)
def _make_msg(H, feat_split):
    rows_per_tile = N // NS
    zrows = 125
    per_tile = EP // NS if feat_split else EP // (NC * NS)

    SB = 2560
    n_sb = per_tile // SB
    sb_chunks = SB // CH

    @functools.partial(
        pl.kernel,
        out_type=jax.ShapeDtypeStruct((NC, N, H), jnp.float32),
        mesh=_mesh(),
        compiler_params=pltpu.CompilerParams(needs_layout_passes=False),
        scratch_types=[
            pltpu.VMEM_SHARED((N, H), jnp.float32),
            pltpu.VMEM((SB,), jnp.int32),
            pltpu.VMEM((sb_chunks, 1, CH), jnp.int32),
            pltpu.VMEM((SB,), jnp.float32),
            pltpu.VMEM((CH, H), jnp.float32),
            pltpu.VMEM((CH, H), jnp.float32),
            pltpu.SemaphoreType.DMA,
            pltpu.SemaphoreType.DMA,
        ],
    )
    def msg(y_hbm, row_hbm, col2_hbm, ew_hbm, out_hbm,
            acc, gidx, colb, ewb, rows0, rows1, gsem0, gsem1):
        c = lax.axis_index("c")
        s = lax.axis_index("s")
        tile0 = s * per_tile if feat_split else (c * NS + s) * per_tile

        def zero(i, carry):
            for j in range(H // 16):
                rows0[i, pl.ds(j * 16, 16)] = _Z16F()
            return carry

        lax.fori_loop(0, zrows, zero, 0)
        for j in range(rows_per_tile // zrows):
            pltpu.sync_copy(
                rows0.at[pl.ds(0, zrows)],
                acc.at[pl.ds(s * rows_per_tile + j * zrows, zrows)])
        plsc.subcore_barrier()

        def gather(k, buf, sem):
            return pltpu.async_copy(
                y_hbm.at[gidx.at[pl.ds(k * CH, CH)]], buf, sem)

        def scale_scatter(k, buf):
            def grp(g, gcarry):
                w = ewb[pl.ds(k * CH + g * 16, 16)]
                for l in range(16):
                    f = w[l]
                    e = g * 16 + l
                    for j in range(H // 16):
                        buf[e, pl.ds(j * 16, 16)] = (
                            buf[e, pl.ds(j * 16, 16)] * f)
                return gcarry

            lax.fori_loop(0, CH // 16, grp, 0)
            pltpu.sync_copy(buf, acc.at[colb.at[k, 0]], add=True)

        def superblock(b, carry):
            off = tile0 + b * SB
            pltpu.sync_copy(row_hbm.at[pl.ds(off, SB)], gidx)
            pltpu.sync_copy(col2_hbm.at[pl.ds(off // CH, sb_chunks)], colb)
            pltpu.sync_copy(ew_hbm.at[pl.ds(off, SB)], ewb)
            if feat_split:
                def mkidx(i, icarry):
                    gidx[pl.ds(i * 16, 16)] = gidx[pl.ds(i * 16, 16)] * 2 + c
                    return icarry

                lax.fori_loop(0, SB // 16, mkidx, 0)
            gather(0, rows0, gsem0)

            def pair(t, pcarry):
                k0 = t * 2
                gather(k0 + 1, rows1, gsem1)
                pltpu.make_async_copy(
                    y_hbm.at[gidx.at[pl.ds(k0 * CH, CH)]], rows0,
                    gsem0).wait()
                scale_scatter(k0, rows0)

                @pl.when(k0 + 2 < sb_chunks)
                def _next():
                    gather(k0 + 2, rows0, gsem0)

                pltpu.make_async_copy(
                    y_hbm.at[gidx.at[pl.ds(k0 * CH, CH)]], rows1,
                    gsem1).wait()
                scale_scatter(k0 + 1, rows1)
                return pcarry

            lax.fori_loop(0, sb_chunks // 2, pair, 0)
            return carry

        lax.fori_loop(0, n_sb, superblock, 0)
        plsc.subcore_barrier()

        @pl.when(s == 0)
        def _copy_out():
            pltpu.sync_copy(acc, out_hbm.at[c])

    return msg


def _tc_first_body(degp_ref, x_ref, w_ref, dis_ref, y_ref):
    deg = jnp.sum(degp_ref[...], axis=1, keepdims=True) + 1.0
    dis = lax.rsqrt(deg)
    dis_ref[...] = dis
    y_ref[...] = jnp.dot(x_ref[...], w_ref[...],
                         preferred_element_type=jnp.float32) * dis


def _combine_body(acc_ref, y_ref, dis_ref, b_ref, g_ref, be_ref, w_ref,
                  out_ref):
    sm = jnp.concatenate([acc_ref[0], acc_ref[1]], axis=1) + y_ref[...]
    dis = dis_ref[...]
    o = sm * dis + b_ref[...]
    mu = jnp.mean(o, axis=0, keepdims=True)
    var = jnp.mean(o * o, axis=0, keepdims=True) - mu * mu
    h = jnp.maximum(
        (o - mu) * lax.rsqrt(var + 1e-5) * g_ref[...] + be_ref[...], 0.0)
    out_ref[...] = jnp.dot(h, w_ref[...],
                           preferred_element_type=jnp.float32) * dis


def _final_body(acc_ref, y_ref, dis_ref, b_ref, g_ref, be_ref,
                wh1_ref, bh1_ref, wh2_ref, bh2_ref, logit_ref, z_ref):
    sm = acc_ref[0] + acc_ref[1] + y_ref[...]
    o = sm * dis_ref[...] + b_ref[...]
    mu = jnp.mean(o, axis=0, keepdims=True)
    var = jnp.mean(o * o, axis=0, keepdims=True) - mu * mu
    z = jnp.maximum(
        (o - mu) * lax.rsqrt(var + 1e-5) * g_ref[...] + be_ref[...], 0.0)
    z_ref[...] = z
    t = jnp.maximum(
        jnp.dot(z, wh1_ref[...], preferred_element_type=jnp.float32)
        + bh1_ref[...], 0.0)
    logit_ref[...] = (
        jnp.dot(t, wh2_ref[...], preferred_element_type=jnp.float32)
        + bh2_ref[...])


def _vmem_call(body, n_in, out_shapes):
    return pl.pallas_call(
        body,
        in_specs=[pl.BlockSpec(memory_space=pltpu.VMEM)] * n_in,
        out_specs=jax.tree.map(
            lambda _: pl.BlockSpec(memory_space=pltpu.VMEM), out_shapes),
        out_shape=out_shapes,
    )


_SDS = jax.ShapeDtypeStruct
_tc_first = _vmem_call(
    _tc_first_body, 3,
    (_SDS((N, 1), jnp.float32), _SDS((N, 256), jnp.float32)))
_tc_comb1 = _vmem_call(_combine_body, 7, _SDS((N, 256), jnp.float32))
_tc_comb2 = _vmem_call(_combine_body, 7, _SDS((N, 128), jnp.float32))
_tc_final = _vmem_call(
    _final_body, 10,
    (_SDS((N, 1), jnp.float32), _SDS((N, 128), jnp.float32)))


def kernel(x, edge_index, edge_weight, W1, b1, g1, be1, W2, b2, g2, be2,
           W3, b3, g3, be3, Wh1, bh1, Wh2, bh2):
    row = edge_index[0]
    col = edge_index[1]
    padi = jnp.zeros((EP - E,), edge_index.dtype)
    row_p = jnp.concatenate([row, padi])
    col_p = jnp.concatenate([col, padi])
    ew_p = jnp.concatenate([edge_weight, jnp.zeros((EP - E,), jnp.float32)])

    degp = _make_deg()(col_p, ew_p)
    dis, y1 = _tc_first(degp.T, x, W1)

    r2 = lambda a: a.reshape(1, -1)
    col2 = col_p.reshape(EP // CH, 1, CH)
    _msg_f = _make_msg(128, True)
    acc1 = _msg_f(y1.reshape(2 * N, 128), row_p, col2, ew_p)
    y2 = _tc_comb1(acc1, y1, dis, r2(b1), r2(g1), r2(be1), W2)
    acc2 = _msg_f(y2.reshape(2 * N, 128), row_p, col2, ew_p)
    y3 = _tc_comb2(acc2, y2, dis, r2(b2), r2(g2), r2(be2), W3)
    acc3 = _make_msg(128, False)(y3, row_p, col2, ew_p)
    logits, z = _tc_final(acc3, y3, dis, r2(b3), r2(g3), r2(be3),
                          Wh1, r2(bh1), Wh2, r2(bh2))
    return logits.reshape(-1), z


# speedup vs baseline: 7.2696x; 1.0407x over previous
import functools

import jax
import jax.numpy as jnp
from jax import lax
from jax.experimental import pallas as pl
from jax.experimental.pallas import tpu as pltpu
from jax.experimental.pallas import tpu_sc as plsc

N = 10000
E = 160000
EP = 163840
NC = 2
NS = 16
MC = 64

@functools.lru_cache(maxsize=None)
def _mesh():
    return plsc.VectorSubcoreMesh(
        core_axis_name="c", subcore_axis_name="s",
        num_cores=NC, num_subcores=NS)


_Z16F = functools.partial(jnp.zeros, (16,), jnp.float32)


@functools.lru_cache(maxsize=None)
def _make_deg():
    return functools.partial(
        pl.kernel,
        out_type=jax.ShapeDtypeStruct((NC * NS, N), jnp.float32),
        mesh=_mesh(),
        compiler_params=pltpu.CompilerParams(needs_layout_passes=False),
        scratch_types=[
            pltpu.VMEM((N,), jnp.float32),
            pltpu.VMEM((EP // (NC * NS),), jnp.int32),
            pltpu.VMEM((EP // (NC * NS),), jnp.float32),
        ],
    )(_deg_body)


def _deg_body(col_hbm, ew_hbm, out_hbm, degl, colb, ewb):
    c = lax.axis_index("c")
    s = lax.axis_index("s")
    wid = s * NC + c
    per_tile = EP // (NC * NS)

    pltpu.sync_copy(col_hbm.at[pl.ds(wid * per_tile, per_tile)], colb)
    pltpu.sync_copy(ew_hbm.at[pl.ds(wid * per_tile, per_tile)], ewb)

    def zero(i, carry):
        degl[pl.ds(i * 16, 16)] = _Z16F()
        return carry

    lax.fori_loop(0, N // 16, zero, 0)

    def grp(j, carry):
        idx = colb[pl.ds(j * 16, 16)]
        w = ewb[pl.ds(j * 16, 16)]
        plsc.addupdate_scatter(degl, [idx], w)
        return carry

    lax.fori_loop(0, per_tile // 16, grp, 0)
    pltpu.sync_copy(degl, out_hbm.at[wid])


@functools.lru_cache(maxsize=None)
def _make_msg(H, feat_split):
    rows_per_tile = N // NS
    zrows = 125
    per_tile = EP // NS if feat_split else EP // (NC * NS)

    SB = 2560
    n_sb = per_tile // SB
    sb_chunks = SB // MC
    NBUF = 4

    @functools.partial(
        pl.kernel,
        out_type=jax.ShapeDtypeStruct((NC, N, H), jnp.float32),
        mesh=_mesh(),
        compiler_params=pltpu.CompilerParams(needs_layout_passes=False),
        scratch_types=(
            [pltpu.VMEM_SHARED((N, H), jnp.float32),
             pltpu.VMEM((SB,), jnp.int32),
             pltpu.VMEM((sb_chunks, 1, MC), jnp.int32),
             pltpu.VMEM((SB,), jnp.float32)]
            + [pltpu.VMEM((MC, H), jnp.float32)] * NBUF
            + [pltpu.SemaphoreType.DMA] * (2 * NBUF)
        ),
    )
    def msg(y_hbm, row_hbm, col2_hbm, ew_hbm, out_hbm,
            acc, gidx, colb, ewb, *bufsem):
        rows = bufsem[:NBUF]
        gsem = bufsem[NBUF:2 * NBUF]
        ssem = bufsem[2 * NBUF:]
        c = lax.axis_index("c")
        s = lax.axis_index("s")
        tile0 = s * per_tile if feat_split else (c * NS + s) * per_tile

        def zero(i, carry):
            for j in range(H // 16):
                rows[0][i, pl.ds(j * 16, 16)] = _Z16F()
                rows[1][i, pl.ds(j * 16, 16)] = _Z16F()
            return carry

        lax.fori_loop(0, MC, zero, 0)
        base = s * rows_per_tile
        for j in range(rows_per_tile // MC):
            pltpu.sync_copy(rows[0], acc.at[pl.ds(base + j * MC, MC)])
        pltpu.sync_copy(rows[1].at[pl.ds(0, rows_per_tile % MC)],
                        acc.at[pl.ds(base + rows_per_tile - rows_per_tile % MC,
                                     rows_per_tile % MC)])
        plsc.subcore_barrier()

        def gather(k, b):
            return pltpu.async_copy(
                y_hbm.at[gidx.at[pl.ds(k * MC, MC)]], rows[b], gsem[b])

        def scale(k, b):
            buf = rows[b]

            def grp(g, gcarry):
                w = ewb[pl.ds(k * MC + g * 16, 16)]
                for l in range(16):
                    f = w[l]
                    e = g * 16 + l
                    for j in range(H // 16):
                        buf[e, pl.ds(j * 16, 16)] = (
                            buf[e, pl.ds(j * 16, 16)] * f)
                return gcarry

            lax.fori_loop(0, MC // 16, grp, 0)

        def superblock(sb, carry):
            off = tile0 + sb * SB
            pltpu.sync_copy(row_hbm.at[pl.ds(off, SB)], gidx)
            pltpu.sync_copy(col2_hbm.at[pl.ds(off // MC, sb_chunks)], colb)
            pltpu.sync_copy(ew_hbm.at[pl.ds(off, SB)], ewb)
            if feat_split:
                def mkidx(i, icarry):
                    gidx[pl.ds(i * 16, 16)] = gidx[pl.ds(i * 16, 16)] * 2 + c
                    return icarry

                lax.fori_loop(0, SB // 16, mkidx, 0)
            gather(0, 0)
            gather(1, 1)

            def quad(t, qcarry):
                for p in range(NBUF):
                    k = t * NBUF + p
                    q = (p + 2) % NBUF
                    pltpu.make_async_copy(
                        y_hbm.at[gidx.at[pl.ds(k * MC, MC)]], rows[p],
                        gsem[p]).wait()
                    scale(k, p)
                    pltpu.make_async_copy(
                        rows[p], acc.at[colb.at[k, 0]],
                        ssem[p]).start(add=True)

                    @pl.when(k >= 2)
                    def _drain():
                        pltpu.make_async_copy(
                            rows[q], acc.at[colb.at[k, 0]], ssem[q]).wait()

                    @pl.when(k + 2 < sb_chunks)
                    def _ahead():
                        gather(k + 2, q)
                return qcarry

            lax.fori_loop(0, sb_chunks // NBUF, quad, 0)
            pltpu.make_async_copy(
                rows[2], acc.at[colb.at[sb_chunks - 2, 0]], ssem[2]).wait()
            pltpu.make_async_copy(
                rows[3], acc.at[colb.at[sb_chunks - 1, 0]], ssem[3]).wait()
            return carry

        lax.fori_loop(0, n_sb, superblock, 0)
        plsc.subcore_barrier()

        @pl.when(s < NS - 1)
        def _copy_out_body():
            pltpu.sync_copy(acc.at[pl.ds(s * 624, 624)],
                            out_hbm.at[c, pl.ds(s * 624, 624)])

        @pl.when(s == NS - 1)
        def _copy_out_tail():
            pltpu.sync_copy(acc.at[pl.ds(624 * (NS - 1), N - 624 * (NS - 1))],
                            out_hbm.at[c, pl.ds(624 * (NS - 1),
                                                N - 624 * (NS - 1))])

    return msg


def _tc_first_body(degp_ref, x_ref, w_ref, dis_ref, y_ref):
    deg = jnp.sum(degp_ref[...], axis=1, keepdims=True) + 1.0
    dis = lax.rsqrt(deg)
    dis_ref[...] = dis
    y_ref[...] = jnp.dot(x_ref[...], w_ref[...],
                         preferred_element_type=jnp.float32) * dis


def _combine_body(acc_ref, y_ref, dis_ref, b_ref, g_ref, be_ref, w_ref,
                  out_ref):
    sm = jnp.concatenate([acc_ref[0], acc_ref[1]], axis=1) + y_ref[...]
    dis = dis_ref[...]
    o = sm * dis + b_ref[...]
    mu = jnp.mean(o, axis=0, keepdims=True)
    var = jnp.mean(o * o, axis=0, keepdims=True) - mu * mu
    h = jnp.maximum(
        (o - mu) * lax.rsqrt(var + 1e-5) * g_ref[...] + be_ref[...], 0.0)
    out_ref[...] = jnp.dot(h, w_ref[...],
                           preferred_element_type=jnp.float32) * dis


def _final_body(acc_ref, y_ref, dis_ref, b_ref, g_ref, be_ref,
                wh1_ref, bh1_ref, wh2_ref, bh2_ref, logit_ref, z_ref):
    sm = acc_ref[0] + acc_ref[1] + y_ref[...]
    o = sm * dis_ref[...] + b_ref[...]
    mu = jnp.mean(o, axis=0, keepdims=True)
    var = jnp.mean(o * o, axis=0, keepdims=True) - mu * mu
    z = jnp.maximum(
        (o - mu) * lax.rsqrt(var + 1e-5) * g_ref[...] + be_ref[...], 0.0)
    z_ref[...] = z
    t = jnp.maximum(
        jnp.dot(z, wh1_ref[...], preferred_element_type=jnp.float32)
        + bh1_ref[...], 0.0)
    logit_ref[...] = (
        jnp.dot(t, wh2_ref[...], preferred_element_type=jnp.float32)
        + bh2_ref[...])


def _vmem_call(body, n_in, out_shapes):
    return pl.pallas_call(
        body,
        in_specs=[pl.BlockSpec(memory_space=pltpu.VMEM)] * n_in,
        out_specs=jax.tree.map(
            lambda _: pl.BlockSpec(memory_space=pltpu.VMEM), out_shapes),
        out_shape=out_shapes,
    )


_SDS = jax.ShapeDtypeStruct
_tc_first = _vmem_call(
    _tc_first_body, 3,
    (_SDS((N, 1), jnp.float32), _SDS((N, 256), jnp.float32)))
_tc_comb1 = _vmem_call(_combine_body, 7, _SDS((N, 256), jnp.float32))
_tc_comb2 = _vmem_call(_combine_body, 7, _SDS((N, 128), jnp.float32))
_tc_final = _vmem_call(
    _final_body, 10,
    (_SDS((N, 1), jnp.float32), _SDS((N, 128), jnp.float32)))


def kernel(x, edge_index, edge_weight, W1, b1, g1, be1, W2, b2, g2, be2,
           W3, b3, g3, be3, Wh1, bh1, Wh2, bh2):
    row = edge_index[0]
    col = edge_index[1]
    padi = jnp.zeros((EP - E,), edge_index.dtype)
    row_p = jnp.concatenate([row, padi])
    col_p = jnp.concatenate([col, padi])
    ew_p = jnp.concatenate([edge_weight, jnp.zeros((EP - E,), jnp.float32)])

    degp = _make_deg()(col_p, ew_p)
    dis, y1 = _tc_first(degp.T, x, W1)

    r2 = lambda a: a.reshape(1, -1)
    col2 = col_p.reshape(EP // MC, 1, MC)
    _msg_f = _make_msg(128, True)
    acc1 = _msg_f(y1.reshape(2 * N, 128), row_p, col2, ew_p)
    y2 = _tc_comb1(acc1, y1, dis, r2(b1), r2(g1), r2(be1), W2)
    acc2 = _msg_f(y2.reshape(2 * N, 128), row_p, col2, ew_p)
    y3 = _tc_comb2(acc2, y2, dis, r2(b2), r2(g2), r2(be2), W3)
    acc3 = _make_msg(128, False)(y3, row_p, col2, ew_p)
    logits, z = _tc_final(acc3, y3, dis, r2(b3), r2(g3), r2(be3),
                          Wh1, r2(bh1), Wh2, r2(bh2))
    return logits.reshape(-1), z
